# baseline (speedup 1.0000x reference)
import functools
import jax
import jax.numpy as jnp
from jax import lax
from jax.experimental import pallas as pl
from jax.experimental.pallas import tpu as pltpu
from jax.experimental.pallas import tpu_sc as plsc

R2 = 0.01
NS = 32
NB = 16384
M = 4096
C = 64
ROW = 3 + C
OSZ = ROW * NS
NQT = 128
QB = 4
NG = NQT // QB
NV = NB // 16


def _sc_body(x_h, y_h, z_h, qx_h, qy_h, qz_h, feat_h,
             out_h, idx_h,
             xv, yv, zv, qxv, qyv, qzv,
             hb0, hb1, hb2, hb3, gidx, fbuf, otile, idxb, sem):
    cid = lax.axis_index("c")
    sid = lax.axis_index("s")
    wid = cid * 16 + sid
    pbase = cid * NB
    qbase = wid * NQT
    pltpu.sync_copy(x_h.at[pl.ds(pbase, NB)], xv)
    pltpu.sync_copy(y_h.at[pl.ds(pbase, NB)], yv)
    pltpu.sync_copy(z_h.at[pl.ds(pbase, NB)], zv)
    pltpu.sync_copy(qx_h.at[pl.ds(qbase, NQT)], qxv.at[pl.ds(0, NQT)])
    pltpu.sync_copy(qy_h.at[pl.ds(qbase, NQT)], qyv.at[pl.ds(0, NQT)])
    pltpu.sync_copy(qz_h.at[pl.ds(qbase, NQT)], qzv.at[pl.ds(0, NQT)])
    lanes = jnp.arange(16, dtype=jnp.int32)
    hbs = [hb0, hb1, hb2, hb3]

    def group(g, carry):
        ql = g * QB
        qxw = qxv[pl.ds(ql, 16)]
        qyw = qyv[pl.ds(ql, 16)]
        qzw = qzv[pl.ds(ql, 16)]
        qxs = [qxw[q] for q in range(QB)]
        qys = [qyw[q] for q in range(QB)]
        qzs = [qzw[q] for q in range(QB)]

        def scan_step(v, st):
            cnts = st
            off = v * 16
            px = xv[pl.ds(off, 16)]
            py = yv[pl.ds(off, 16)]
            pz = zv[pl.ds(off, 16)]
            cand = lanes + off
            new = []
            for q in range(QB):
                dx = px - qxs[q]
                dy = py - qys[q]
                dz = pz - qzs[q]
                d2 = dx * dx + dy * dy + dz * dz
                m = d2 < R2
                plsc.store_compressed(hbs[q].at[pl.ds(cnts[q], 16)], cand,
                                      mask=m)
                pc = plsc.all_reduce_population_count(m)[0]
                new.append(jnp.minimum(cnts[q] + pc, NS))
            return tuple(new)

        cnts = lax.fori_loop(0, NV, scan_step,
                             tuple(jnp.int32(0) for _ in range(QB)))

        zms = []
        for q in range(QB):
            cnt = cnts[q]
            hb = hbs[q]
            i0 = hb[pl.ds(0, 16)]
            i1 = hb[pl.ds(16, 16)]
            first = jnp.where(cnt == 0, jnp.int32(0), i0[0])
            i0 = jnp.where(lanes < cnt, i0, first)
            i1 = jnp.where(lanes + 16 < cnt, i1, first)
            idxb[pl.ds(q * NS, 16)] = i0
            idxb[pl.ds(q * NS + 16, 16)] = i1
            gidx[pl.ds(q * NS, 16)] = i0 + pbase
            gidx[pl.ds(q * NS + 16, 16)] = i1 + pbase
            zms.append(jnp.where(cnt == 0, jnp.float32(0), jnp.float32(1)))

        mq0 = qbase + ql
        pltpu.sync_copy(idxb, idx_h.at[pl.ds(mq0 * NS, QB * NS)])
        cp = pltpu.async_copy(feat_h.at[gidx], fbuf, sem)

        for q in range(QB):
            zm = zms[q]
            i0 = idxb[pl.ds(q * NS, 16)]
            i1 = idxb[pl.ds(q * NS + 16, 16)]
            ob = q * OSZ
            gx0 = plsc.load_gather(xv, [i0])
            gx1 = plsc.load_gather(xv, [i1])
            gy0 = plsc.load_gather(yv, [i0])
            gy1 = plsc.load_gather(yv, [i1])
            gz0 = plsc.load_gather(zv, [i0])
            gz1 = plsc.load_gather(zv, [i1])
            otile[pl.ds(ob + 0, 16)] = (gx0 - qxs[q]) * zm
            otile[pl.ds(ob + 16, 16)] = (gx1 - qxs[q]) * zm
            otile[pl.ds(ob + 32, 16)] = (gy0 - qys[q]) * zm
            otile[pl.ds(ob + 48, 16)] = (gy1 - qys[q]) * zm
            otile[pl.ds(ob + 64, 16)] = (gz0 - qzs[q]) * zm
            otile[pl.ds(ob + 80, 16)] = (gz1 - qzs[q]) * zm

        cp.wait()

        for q in range(QB):
            zm = zms[q]
            r0 = lanes + q * NS
            r1 = r0 + 16
            qob = q * OSZ + 3 * NS

            def chan(ch, carry2):
                colv = jnp.zeros((16,), jnp.int32) + ch
                fa = plsc.load_gather(fbuf, [r0, colv])
                fb = plsc.load_gather(fbuf, [r1, colv])
                base = qob + ch * NS
                otile[pl.ds(base, 16)] = fa * zm
                otile[pl.ds(base + 16, 16)] = fb * zm
                return carry2

            lax.fori_loop(0, C, chan, 0)

        pltpu.sync_copy(otile, out_h.at[pl.ds(mq0 * OSZ, QB * OSZ)])
        return carry

    lax.fori_loop(0, NG, group, 0)


def _make_call():
    mesh = plsc.VectorSubcoreMesh(core_axis_name="c", subcore_axis_name="s")
    return pl.kernel(
        _sc_body,
        out_type=[
            jax.ShapeDtypeStruct((M * OSZ,), jnp.float32),
            jax.ShapeDtypeStruct((M * NS,), jnp.int32),
        ],
        mesh=mesh,
        compiler_params=pltpu.CompilerParams(
            needs_layout_passes=False, use_tc_tiling_on_sc=False),
        scratch_types=[
            pltpu.VMEM((NB,), jnp.float32),
            pltpu.VMEM((NB,), jnp.float32),
            pltpu.VMEM((NB,), jnp.float32),
            pltpu.VMEM((NQT + 16,), jnp.float32),
            pltpu.VMEM((NQT + 16,), jnp.float32),
            pltpu.VMEM((NQT + 16,), jnp.float32),
            pltpu.VMEM((64,), jnp.int32),
            pltpu.VMEM((64,), jnp.int32),
            pltpu.VMEM((64,), jnp.int32),
            pltpu.VMEM((64,), jnp.int32),
            pltpu.VMEM((QB * NS,), jnp.int32),
            pltpu.VMEM((QB * NS, C), jnp.float32),
            pltpu.VMEM((QB * OSZ,), jnp.float32),
            pltpu.VMEM((QB * NS,), jnp.int32),
            pltpu.SemaphoreType.DMA,
        ],
    )


@jax.jit
def kernel(xyz, xyz_batch_cnt, new_xyz, new_xyz_batch_cnt, features):
    xyz_t = xyz.T
    new_t = new_xyz.T
    out_flat, idx = _make_call()(
        xyz_t[0], xyz_t[1], xyz_t[2],
        new_t[0], new_t[1], new_t[2],
        features,
    )
    return out_flat.reshape(M, ROW, NS), idx.reshape(M, NS)


# speedup vs baseline: 1.3386x; 1.3386x over previous
import functools
import jax
import jax.numpy as jnp
from jax import lax
from jax.experimental import pallas as pl
from jax.experimental.pallas import tpu as pltpu
from jax.experimental.pallas import tpu_sc as plsc

R2 = 0.01
NS = 32
NB = 16384
M = 4096
C = 64
ROW = 3 + C
OSZ = ROW * NS
NQT = 128
QB = 4
NG = NQT // QB
NV = NB // 16


def _sc_body(x_h, y_h, z_h, qx_h, qy_h, qz_h, feat_h,
             out_h, idx_h,
             xv, yv, zv, qxv, qyv, qzv,
             hb0, hb1, hb2, hb3, gidx, fbuf, otile, idxb, sem):
    cid = lax.axis_index("c")
    sid = lax.axis_index("s")
    wid = cid * 16 + sid
    pbase = cid * NB
    qbase = wid * NQT
    pltpu.sync_copy(x_h.at[pl.ds(pbase, NB)], xv)
    pltpu.sync_copy(y_h.at[pl.ds(pbase, NB)], yv)
    pltpu.sync_copy(z_h.at[pl.ds(pbase, NB)], zv)
    pltpu.sync_copy(qx_h.at[pl.ds(qbase, NQT)], qxv.at[pl.ds(0, NQT)])
    pltpu.sync_copy(qy_h.at[pl.ds(qbase, NQT)], qyv.at[pl.ds(0, NQT)])
    pltpu.sync_copy(qz_h.at[pl.ds(qbase, NQT)], qzv.at[pl.ds(0, NQT)])
    lanes = jnp.arange(16, dtype=jnp.int32)
    hbs = [hb0, hb1, hb2, hb3]

    def group(g, carry):
        ql = g * QB
        qxw = qxv[pl.ds(ql, 16)]
        qyw = qyv[pl.ds(ql, 16)]
        qzw = qzv[pl.ds(ql, 16)]
        qxs = [qxw[q] for q in range(QB)]
        qys = [qyw[q] for q in range(QB)]
        qzs = [qzw[q] for q in range(QB)]

        def scan_step(v, st):
            cnts = st
            off = v * 16
            px = xv[pl.ds(off, 16)]
            py = yv[pl.ds(off, 16)]
            pz = zv[pl.ds(off, 16)]
            cand = lanes + off
            new = []
            for q in range(QB):
                dx = px - qxs[q]
                dy = py - qys[q]
                dz = pz - qzs[q]
                d2 = dx * dx + dy * dy + dz * dz
                m = d2 < R2
                plsc.store_compressed(hbs[q].at[pl.ds(cnts[q], 16)], cand,
                                      mask=m)
                pc = jnp.sum(m.astype(jnp.int32))
                new.append(jnp.minimum(cnts[q] + pc, NS))
            return tuple(new)

        BLK = 64

        def blk_cond(st):
            v = st[0]
            full = st[1] >= NS
            for q in range(2, QB + 1):
                full = full & (st[q] >= NS)
            return (v < NV) & (~full)

        def blk_step(st):
            v = st[0]
            cnts = lax.fori_loop(v, v + BLK, scan_step, st[1:])
            return (v + BLK,) + tuple(cnts)

        st = lax.while_loop(blk_cond, blk_step,
                            tuple(jnp.int32(0) for _ in range(QB + 1)))
        cnts = st[1:]

        zms = []
        for q in range(QB):
            cnt = cnts[q]
            hb = hbs[q]
            i0 = hb[pl.ds(0, 16)]
            i1 = hb[pl.ds(16, 16)]
            first = jnp.where(cnt == 0, jnp.int32(0), i0[0])
            i0 = jnp.where(lanes < cnt, i0, first)
            i1 = jnp.where(lanes + 16 < cnt, i1, first)
            idxb[pl.ds(q * NS, 16)] = i0
            idxb[pl.ds(q * NS + 16, 16)] = i1
            gidx[pl.ds(q * NS, 16)] = i0 + pbase
            gidx[pl.ds(q * NS + 16, 16)] = i1 + pbase
            zms.append(jnp.where(cnt == 0, jnp.float32(0), jnp.float32(1)))

        mq0 = qbase + ql
        pltpu.sync_copy(idxb, idx_h.at[pl.ds(mq0 * NS, QB * NS)])
        cp = pltpu.async_copy(feat_h.at[gidx], fbuf, sem)

        for q in range(QB):
            zm = zms[q]
            i0 = idxb[pl.ds(q * NS, 16)]
            i1 = idxb[pl.ds(q * NS + 16, 16)]
            ob = q * OSZ
            gx0 = plsc.load_gather(xv, [i0])
            gx1 = plsc.load_gather(xv, [i1])
            gy0 = plsc.load_gather(yv, [i0])
            gy1 = plsc.load_gather(yv, [i1])
            gz0 = plsc.load_gather(zv, [i0])
            gz1 = plsc.load_gather(zv, [i1])
            otile[pl.ds(ob + 0, 16)] = (gx0 - qxs[q]) * zm
            otile[pl.ds(ob + 16, 16)] = (gx1 - qxs[q]) * zm
            otile[pl.ds(ob + 32, 16)] = (gy0 - qys[q]) * zm
            otile[pl.ds(ob + 48, 16)] = (gy1 - qys[q]) * zm
            otile[pl.ds(ob + 64, 16)] = (gz0 - qzs[q]) * zm
            otile[pl.ds(ob + 80, 16)] = (gz1 - qzs[q]) * zm

        cp.wait()

        for q in range(QB):
            zm = zms[q]
            r0 = lanes + q * NS
            r1 = r0 + 16
            qob = q * OSZ + 3 * NS

            def chan(ch, carry2):
                colv = jnp.zeros((16,), jnp.int32) + ch
                fa = plsc.load_gather(fbuf, [r0, colv])
                fb = plsc.load_gather(fbuf, [r1, colv])
                base = qob + ch * NS
                otile[pl.ds(base, 16)] = fa * zm
                otile[pl.ds(base + 16, 16)] = fb * zm
                return carry2

            lax.fori_loop(0, C, chan, 0)

        pltpu.sync_copy(otile, out_h.at[pl.ds(mq0 * OSZ, QB * OSZ)])
        return carry

    lax.fori_loop(0, NG, group, 0)


def _make_call():
    mesh = plsc.VectorSubcoreMesh(core_axis_name="c", subcore_axis_name="s")
    return pl.kernel(
        _sc_body,
        out_type=[
            jax.ShapeDtypeStruct((M * OSZ,), jnp.float32),
            jax.ShapeDtypeStruct((M * NS,), jnp.int32),
        ],
        mesh=mesh,
        compiler_params=pltpu.CompilerParams(
            needs_layout_passes=False, use_tc_tiling_on_sc=False),
        scratch_types=[
            pltpu.VMEM((NB,), jnp.float32),
            pltpu.VMEM((NB,), jnp.float32),
            pltpu.VMEM((NB,), jnp.float32),
            pltpu.VMEM((NQT + 16,), jnp.float32),
            pltpu.VMEM((NQT + 16,), jnp.float32),
            pltpu.VMEM((NQT + 16,), jnp.float32),
            pltpu.VMEM((64,), jnp.int32),
            pltpu.VMEM((64,), jnp.int32),
            pltpu.VMEM((64,), jnp.int32),
            pltpu.VMEM((64,), jnp.int32),
            pltpu.VMEM((QB * NS,), jnp.int32),
            pltpu.VMEM((QB * NS, C), jnp.float32),
            pltpu.VMEM((QB * OSZ,), jnp.float32),
            pltpu.VMEM((QB * NS,), jnp.int32),
            pltpu.SemaphoreType.DMA,
        ],
    )


@jax.jit
def kernel(xyz, xyz_batch_cnt, new_xyz, new_xyz_batch_cnt, features):
    xyz_t = xyz.T
    new_t = new_xyz.T
    out_flat, idx = _make_call()(
        xyz_t[0], xyz_t[1], xyz_t[2],
        new_t[0], new_t[1], new_t[2],
        features,
    )
    return out_flat.reshape(M, ROW, NS), idx.reshape(M, NS)
